# speedup vs baseline: 1.0106x; 1.0106x over previous
import functools

import jax
import jax.numpy as jnp
from jax import lax
from jax.experimental import pallas as pl
from jax.experimental.pallas import tpu as pltpu
from jax.experimental.pallas import tpu_sc as plsc

N = 100000
D = 128
B = 256

NC = 2
NS = 16
NW = NC * NS

C = 80
NCHUNK = N // C
ITERS = (NCHUNK + NW - 1) // NW

_mesh = plsc.VectorSubcoreMesh(core_axis_name="c", subcore_axis_name="s")


@functools.partial(
    pl.kernel,
    out_type=jax.ShapeDtypeStruct((NC, B, D), jnp.float32),
    mesh=_mesh,
    scratch_types=[
        pltpu.VMEM((C,), jnp.int32),
        pltpu.VMEM((C, D), jnp.float32),
        pltpu.VMEM_SHARED((B, D), jnp.float32),
    ],
)
def _sc_pool(h_hbm, batch_hbm, zero_hbm, out_hbm, idx_v, hbuf, acc):
    c = lax.axis_index("c")
    s = lax.axis_index("s")
    wid = s * NC + c

    @pl.when(s == 0)
    def _init():
        pltpu.sync_copy(zero_hbm, acc)

    plsc.subcore_barrier()

    def body(i, carry):
        cid = i * NW + wid

        @pl.when(cid < NCHUNK)
        def _():
            base = cid * C
            pltpu.sync_copy(batch_hbm.at[pl.ds(base, C)], idx_v)
            pltpu.sync_copy(h_hbm.at[pl.ds(base, C), :], hbuf)
            pltpu.sync_copy(hbuf, acc.at[idx_v], add=True)

        return carry

    lax.fori_loop(0, ITERS, body, 0)
    plsc.subcore_barrier()

    @pl.when(s == 0)
    def _flush():
        pltpu.sync_copy(acc, out_hbm.at[c])


@functools.partial(
    pl.kernel,
    out_type=jax.ShapeDtypeStruct((N, D), jnp.float32),
    mesh=_mesh,
    scratch_types=[
        pltpu.VMEM((C,), jnp.int32),
        pltpu.VMEM((C, D), jnp.float32),
        pltpu.VMEM((C, D), jnp.float32),
        pltpu.SemaphoreType.DMA,
    ],
)
def _sc_broadcast(h_hbm, batch_hbm, vn_hbm, out_hbm, idx_v, hbuf, vbuf, sem):
    c = lax.axis_index("c")
    s = lax.axis_index("s")
    wid = s * NC + c

    def body(i, carry):
        cid = i * NW + wid

        @pl.when(cid < NCHUNK)
        def _():
            base = cid * C
            pltpu.sync_copy(batch_hbm.at[pl.ds(base, C)], idx_v)
            pltpu.sync_copy(h_hbm.at[pl.ds(base, C), :], hbuf)
            pltpu.async_copy(vn_hbm.at[idx_v], vbuf, sem).wait()

            def row(r, rc):
                for k in range(D // 16):
                    col = pl.ds(k * 16, 16)
                    hbuf[r, col] = hbuf[r, col] + vbuf[r, col]
                return rc

            lax.fori_loop(0, C, row, 0)
            pltpu.sync_copy(hbuf, out_hbm.at[pl.ds(base, C), :])

        return carry

    lax.fori_loop(0, ITERS, body, 0)


def _fc_body(p_ref, v_ref, w_ref, b_ref, o_ref):
    vn = v_ref[...]
    z = vn + p_ref[0] + p_ref[1]
    y = jnp.dot(z, w_ref[...], preferred_element_type=jnp.float32) + b_ref[...]
    o_ref[...] = vn + jnp.maximum(y, 0.0)


_fc = pl.pallas_call(
    _fc_body,
    out_shape=jax.ShapeDtypeStruct((B, D), jnp.float32),
)


def kernel(h, vn_h, batch, W, b):
    batch_i = batch.astype(jnp.int32)
    zero = jnp.zeros((B, D), jnp.float32)
    pool2 = _sc_pool(h, batch_i, zero)
    vn_new = _fc(pool2, vn_h, W, b.reshape(1, D))
    h_out = _sc_broadcast(h, batch_i, vn_new)
    return h_out, vn_new


# speedup vs baseline: 1.6257x; 1.6087x over previous
import functools

import jax
import jax.numpy as jnp
from jax import lax
from jax.experimental import pallas as pl
from jax.experimental.pallas import tpu as pltpu
from jax.experimental.pallas import tpu_sc as plsc

N = 100000
D = 128
B = 256

NC = 2
NS = 16
NW = NC * NS

G = 100

S1 = 400
NCH1 = N // S1
IT1 = -(-NCH1 // NW)
J1 = S1 // G

S3 = 200
NCH3 = N // S3
IT3 = -(-NCH3 // NW)
J3 = S3 // G

_mesh = plsc.VectorSubcoreMesh(core_axis_name="c", subcore_axis_name="s")


def _issue_loads(h_hbm, bat_hbm, cid, rows, hb, ibs, sem):
    for j in range(len(ibs)):
        pltpu.async_copy(bat_hbm.at[cid, j], ibs[j], sem)
    pltpu.async_copy(h_hbm.at[pl.ds(cid * rows, rows), :], hb, sem)


def _wait_loads(h_hbm, bat_hbm, cid, rows, hb, ibs, sem):
    for j in range(len(ibs)):
        pltpu.make_async_copy(bat_hbm.at[cid, j], ibs[j], sem).wait()
    pltpu.make_async_copy(h_hbm.at[pl.ds(cid * rows, rows), :], hb, sem).wait()


@functools.partial(
    pl.kernel,
    out_type=jax.ShapeDtypeStruct((NC, B, D), jnp.float32),
    mesh=_mesh,
    scratch_types=[
        (pltpu.VMEM((G,), jnp.int32),) * J1,
        (pltpu.VMEM((G,), jnp.int32),) * J1,
        pltpu.VMEM((S1, D), jnp.float32),
        pltpu.VMEM((S1, D), jnp.float32),
        pltpu.VMEM_SHARED((B, D), jnp.float32),
        pltpu.SemaphoreType.DMA,
        pltpu.SemaphoreType.DMA,
    ],
)
def _sc_pool(h_hbm, bat_hbm, zero_hbm, out_hbm,
             ib0, ib1, hb0, hb1, acc, semL0, semL1):
    c = lax.axis_index("c")
    s = lax.axis_index("s")
    wid = s * NC + c

    @pl.when(s == 0)
    def _init():
        pltpu.sync_copy(zero_hbm, acc)

    plsc.subcore_barrier()

    _issue_loads(h_hbm, bat_hbm, wid, S1, hb0, ib0, semL0)

    def _scatter(hb, ibs):
        for j in range(J1):
            pltpu.sync_copy(hb.at[pl.ds(j * G, G), :], acc.at[ibs[j]],
                            add=True)

    def body(k, carry):
        c0 = (2 * k) * NW + wid
        c1 = c0 + NW
        c2 = c0 + 2 * NW

        @pl.when(c0 < NCH1)
        def _slot0():
            _wait_loads(h_hbm, bat_hbm, c0, S1, hb0, ib0, semL0)

            @pl.when(c1 < NCH1)
            def _():
                _issue_loads(h_hbm, bat_hbm, c1, S1, hb1, ib1, semL1)

            _scatter(hb0, ib0)

        @pl.when(c1 < NCH1)
        def _slot1():
            _wait_loads(h_hbm, bat_hbm, c1, S1, hb1, ib1, semL1)

            @pl.when(c2 < NCH1)
            def _():
                _issue_loads(h_hbm, bat_hbm, c2, S1, hb0, ib0, semL0)

            _scatter(hb1, ib1)

        return carry

    lax.fori_loop(0, -(-IT1 // 2), body, 0)
    plsc.subcore_barrier()

    @pl.when(s == 0)
    def _flush():
        pltpu.sync_copy(acc, out_hbm.at[c])


NB = 4
KIT3 = IT3 // NB


@functools.partial(
    pl.kernel,
    out_type=jax.ShapeDtypeStruct((N, D), jnp.float32),
    mesh=_mesh,
    scratch_types=[
        ((pltpu.VMEM((G,), jnp.int32),) * J3,) * NB,
        (pltpu.VMEM((S3, D), jnp.float32),) * NB,
        (pltpu.SemaphoreType.DMA,) * NB,
        (pltpu.SemaphoreType.DMA,) * NB,
        pltpu.SemaphoreType.DMA,
    ],
)
def _sc_broadcast(h_hbm, bat_hbm, vn_hbm, out_hbm,
                  ibs, hbs, semL, semO, semG):
    c = lax.axis_index("c")
    s = lax.axis_index("s")
    wid = s * NC + c

    _issue_loads(h_hbm, bat_hbm, wid, S3, hbs[0], ibs[0], semL[0])

    def section(k, p, first):
        cid = (NB * k + p) * NW + wid
        cn = cid + NW
        q = (p + 1) % NB

        @pl.when(cid < NCH3)
        def _():
            _wait_loads(h_hbm, bat_hbm, cid, S3, hbs[p], ibs[p], semL[p])

            @pl.when(cn < NCH3)
            def _():
                if not (first and p < NB - 1):
                    pltpu.make_async_copy(
                        hbs[q], out_hbm.at[pl.ds(0, S3), :], semO[q]).wait()
                _issue_loads(h_hbm, bat_hbm, cn, S3, hbs[q], ibs[q], semL[q])

            for j in range(J3):
                pltpu.async_copy(vn_hbm.at[ibs[p][j]],
                                 hbs[p].at[pl.ds(j * G, G), :], semG,
                                 add=True)
            for j in range(J3):
                pltpu.make_async_copy(vn_hbm.at[ibs[p][j]],
                                      hbs[p].at[pl.ds(j * G, G), :],
                                      semG).wait()
            pltpu.async_copy(hbs[p], out_hbm.at[pl.ds(cid * S3, S3), :],
                             semO[p])

    for p in range(NB):
        section(0, p, True)

    def body(k, carry):
        for p in range(NB):
            section(k, p, False)
        return carry

    lax.fori_loop(1, KIT3, body, 0)

    for p in range(NB):
        pltpu.make_async_copy(hbs[p], out_hbm.at[pl.ds(0, S3), :],
                              semO[p]).wait()


def _fc_body(p_ref, v_ref, w_ref, b_ref, o_ref):
    vn = v_ref[...]
    z = vn + p_ref[0] + p_ref[1]
    y = jnp.dot(z, w_ref[...], preferred_element_type=jnp.float32) + b_ref[...]
    o_ref[...] = vn + jnp.maximum(y, 0.0)


_fc = pl.pallas_call(
    _fc_body,
    out_shape=jax.ShapeDtypeStruct((B, D), jnp.float32),
)


def kernel(h, vn_h, batch, W, b):
    batch_i = batch.astype(jnp.int32)
    bat1 = batch_i.reshape(NCH1, J1, G)
    bat3 = batch_i.reshape(NCH3, J3, G)
    zero = jnp.zeros((B, D), jnp.float32)
    pool2 = _sc_pool(h, bat1, zero)
    vn_new = _fc(pool2, vn_h, W, b.reshape(1, D))
    h_out = _sc_broadcast(h, bat3, vn_new)
    return h_out, vn_new
